# Initial kernel scaffold; baseline (speedup 1.0000x reference)
import jax
import jax.numpy as jnp
from jax.experimental import pallas as pl


def kernel(xyz, features, W1, g1, b1, W2, g2, b2, W3, g3, b3):
    raise NotImplementedError("write your pallas kernel here")


# speedup vs baseline: 12.3971x; 12.3971x over previous
import functools

import jax
import jax.numpy as jnp
from jax import lax
from jax.experimental import pallas as pl
from jax.experimental.pallas import tpu as pltpu
from jax.experimental.pallas import tpu_sc as plsc

B = 4
N = 8192
C_IN = 64
NPOINT = 1024
NSAMPLE = 32
RADIUS = 0.12
EPS = 1e-5
M_ROWS = B * NPOINT * NSAMPLE
D_PAD = 128
ROW_TILE = 512
N_TILES = M_ROWS // ROW_TILE
CT_TILE = 256


def _fps_body(xyz_ref, newxyz_ref, dists_ref):
    X = xyz_ref[:, 0, :]
    Y = xyz_ref[:, 1, :]
    Z = xyz_ref[:, 2, :]
    iota = lax.broadcasted_iota(jnp.int32, (B, N), 1)
    dists_ref[...] = jnp.full((B, N), 1e10, jnp.float32)

    def coords_of(last):
        mask = iota == last
        lx = jnp.sum(jnp.where(mask, X, 0.0), axis=1, keepdims=True)
        ly = jnp.sum(jnp.where(mask, Y, 0.0), axis=1, keepdims=True)
        lz = jnp.sum(jnp.where(mask, Z, 0.0), axis=1, keepdims=True)
        return lx, ly, lz

    def body(i, last):
        lx, ly, lz = coords_of(last)
        newxyz_ref[:, pl.ds(i - 1, 1), :] = jnp.concatenate(
            [lx[:, :, None], ly[:, :, None], lz[:, :, None]], axis=2)
        d = (X - lx) ** 2 + (Y - ly) ** 2 + (Z - lz) ** 2
        nd = jnp.minimum(dists_ref[...], d)
        dists_ref[...] = nd
        m = jnp.max(nd, axis=1, keepdims=True)
        nxt = jnp.min(jnp.where(nd == m, iota, N), axis=1, keepdims=True)
        return nxt.astype(jnp.int32)

    last = lax.fori_loop(1, NPOINT, body, jnp.zeros((B, 1), jnp.int32))
    lx, ly, lz = coords_of(last)
    newxyz_ref[:, NPOINT - 1:NPOINT, :] = jnp.concatenate(
        [lx[:, :, None], ly[:, :, None], lz[:, :, None]], axis=2)


def _fps(xyz_t):
    return pl.pallas_call(
        _fps_body,
        out_shape=jax.ShapeDtypeStruct((B, NPOINT, 3), jnp.float32),
        scratch_shapes=[pltpu.VMEM((B, N), jnp.float32)],
    )(xyz_t)


def _ballq_body(xyz_ref, c_ref, idx_ref, key_ref):
    t = pl.program_id(0)
    b = t // (NPOINT // CT_TILE)
    P = xyz_ref[0]
    px = xyz_ref[0, 0:1, :]
    py = xyz_ref[0, 1:2, :]
    pz = xyz_ref[0, 2:3, :]
    pn = px * px + py * py + pz * pz
    Ct = c_ref[0]
    cn = jnp.sum(Ct * Ct, axis=1, keepdims=True)
    dot = lax.dot_general(Ct, P, (((1,), (0,)), ((), ())),
                          preferred_element_type=jnp.float32)
    d2 = (cn + pn) - 2.0 * dot
    iota = lax.broadcasted_iota(jnp.int32, (CT_TILE, N), 1)
    r2 = jnp.float32(RADIUS * RADIUS)
    key_ref[...] = jnp.where(d2 < r2, iota, N)
    liota = lax.broadcasted_iota(jnp.int32, (CT_TILE, NSAMPLE), 1)
    m0 = jnp.min(key_ref[...], axis=1, keepdims=True)
    acc0 = jnp.where(liota == 0, m0, N)

    def body(j, carry):
        acc, m = carry
        key = key_ref[...]
        m = jnp.min(jnp.where(key > m, key, N), axis=1, keepdims=True)
        return jnp.where(liota == j, m, acc), m

    acc, _ = lax.fori_loop(1, NSAMPLE, body, (acc0, m0))
    idx_ref[...] = jnp.where(acc == N, acc[:, 0:1], acc) + b * N


def _ballq(xyz_t, new_xyz):
    n_ct = NPOINT // CT_TILE
    return pl.pallas_call(
        _ballq_body,
        grid=(B * n_ct,),
        in_specs=[
            pl.BlockSpec((1, 3, N), lambda t: (t // n_ct, 0, 0)),
            pl.BlockSpec((1, CT_TILE, 3), lambda t: (t // n_ct, t % n_ct, 0)),
        ],
        out_specs=pl.BlockSpec((CT_TILE, NSAMPLE), lambda t: (t, 0)),
        out_shape=jax.ShapeDtypeStruct((B * NPOINT, NSAMPLE), jnp.int32),
        scratch_shapes=[pltpu.VMEM((CT_TILE, N), jnp.int32)],
    )(xyz_t, new_xyz)


def _gather_rows(table, idx_flat):
    info = plsc.get_sparse_core_info()
    nw = info.num_cores * info.num_subcores
    rows_per_w = M_ROWS // nw
    chunk = 512
    n_ch = rows_per_w // chunk
    mesh = plsc.VectorSubcoreMesh(core_axis_name="c", subcore_axis_name="s")

    @functools.partial(
        pl.kernel, mesh=mesh,
        out_type=jax.ShapeDtypeStruct((M_ROWS, D_PAD), jnp.float32),
        scratch_types=[
            pltpu.VMEM((chunk,), jnp.int32),
            pltpu.VMEM((chunk, D_PAD), jnp.float32),
            pltpu.SemaphoreType.DMA,
        ],
    )
    def k(table_hbm, idx_hbm, out_hbm, idx_v, rows_v, sem):
        wid = lax.axis_index("s") * info.num_cores + lax.axis_index("c")
        base = wid * rows_per_w

        def body(j, carry):
            off = base + j * chunk
            pltpu.sync_copy(idx_hbm.at[pl.ds(off, chunk)], idx_v)
            pltpu.async_copy(table_hbm.at[idx_v], rows_v, sem).wait()
            pltpu.sync_copy(rows_v, out_hbm.at[pl.ds(off, chunk)])
            return carry

        lax.fori_loop(0, n_ch, body, 0)

    return k(table, idx_flat)


def _stats_update(st_ref, y):
    t = pl.program_id(0)

    @pl.when(t == 0)
    def _():
        st_ref[...] = jnp.zeros_like(st_ref)

    s = jnp.sum(y, axis=0, keepdims=True)
    ss = jnp.sum(y * y, axis=0, keepdims=True)
    st_ref[...] += jnp.concatenate([s, ss], axis=0)


def _p1(X, C8, W1p, Wx8, c_out):
    def body(x_ref, c_ref, w_ref, wx_ref, y_ref, st_ref):
        y = jnp.dot(x_ref[...], w_ref[...], preferred_element_type=jnp.float32)
        y = y - jnp.dot(c_ref[...], wx_ref[...],
                        preferred_element_type=jnp.float32)
        y_ref[...] = y
        _stats_update(st_ref, y)

    return pl.pallas_call(
        body,
        grid=(N_TILES,),
        in_specs=[
            pl.BlockSpec((ROW_TILE, D_PAD), lambda t: (t, 0)),
            pl.BlockSpec((ROW_TILE, 8), lambda t: (t, 0)),
            pl.BlockSpec((D_PAD, c_out), lambda t: (0, 0)),
            pl.BlockSpec((8, c_out), lambda t: (0, 0)),
        ],
        out_specs=[
            pl.BlockSpec((ROW_TILE, c_out), lambda t: (t, 0)),
            pl.BlockSpec((2, c_out), lambda t: (0, 0)),
        ],
        out_shape=[
            jax.ShapeDtypeStruct((M_ROWS, c_out), jnp.float32),
            jax.ShapeDtypeStruct((2, c_out), jnp.float32),
        ],
    )(X, C8, W1p, Wx8)


def _p2(Y, a, c, Wt, c_in, c_out):
    def body(y_ref, a_ref, c_ref, w_ref, o_ref, st_ref):
        z = jnp.maximum(y_ref[...] * a_ref[...] + c_ref[...], 0.0)
        y = jnp.dot(z, w_ref[...], preferred_element_type=jnp.float32)
        o_ref[...] = y
        _stats_update(st_ref, y)

    return pl.pallas_call(
        body,
        grid=(N_TILES,),
        in_specs=[
            pl.BlockSpec((ROW_TILE, c_in), lambda t: (t, 0)),
            pl.BlockSpec((1, c_in), lambda t: (0, 0)),
            pl.BlockSpec((1, c_in), lambda t: (0, 0)),
            pl.BlockSpec((c_in, c_out), lambda t: (0, 0)),
        ],
        out_specs=[
            pl.BlockSpec((ROW_TILE, c_out), lambda t: (t, 0)),
            pl.BlockSpec((2, c_out), lambda t: (0, 0)),
        ],
        out_shape=[
            jax.ShapeDtypeStruct((M_ROWS, c_out), jnp.float32),
            jax.ShapeDtypeStruct((2, c_out), jnp.float32),
        ],
    )(Y, a, c, Wt)


def _p3(Y, a, c, Wt, c_in, c_out):
    groups = ROW_TILE // NSAMPLE

    def body(y_ref, a_ref, c_ref, w_ref, mx_ref, mn_ref, st_ref):
        z = jnp.maximum(y_ref[...] * a_ref[...] + c_ref[...], 0.0)
        y = jnp.dot(z, w_ref[...], preferred_element_type=jnp.float32)
        _stats_update(st_ref, y)
        mxs, mns = [], []
        for g in range(groups):
            blk = y[g * NSAMPLE:(g + 1) * NSAMPLE]
            mxs.append(jnp.max(blk, axis=0, keepdims=True))
            mns.append(jnp.min(blk, axis=0, keepdims=True))
        mx_ref[...] = jnp.concatenate(mxs, axis=0)
        mn_ref[...] = jnp.concatenate(mns, axis=0)

    n_grp = B * NPOINT
    return pl.pallas_call(
        body,
        grid=(N_TILES,),
        in_specs=[
            pl.BlockSpec((ROW_TILE, c_in), lambda t: (t, 0)),
            pl.BlockSpec((1, c_in), lambda t: (0, 0)),
            pl.BlockSpec((1, c_in), lambda t: (0, 0)),
            pl.BlockSpec((c_in, c_out), lambda t: (0, 0)),
        ],
        out_specs=[
            pl.BlockSpec((groups, c_out), lambda t: (t, 0)),
            pl.BlockSpec((groups, c_out), lambda t: (t, 0)),
            pl.BlockSpec((2, c_out), lambda t: (0, 0)),
        ],
        out_shape=[
            jax.ShapeDtypeStruct((n_grp, c_out), jnp.float32),
            jax.ShapeDtypeStruct((n_grp, c_out), jnp.float32),
            jax.ShapeDtypeStruct((2, c_out), jnp.float32),
        ],
    )(Y, a, c, Wt)


def _p4(mx, mn, a, c, c_out):
    tile = 512
    n_t = (B * NPOINT) // tile
    per_b = NPOINT // tile

    def body(mx_ref, mn_ref, a_ref, c_ref, o_ref):
        av = a_ref[...]
        pick = jnp.where(av >= 0.0, mx_ref[...], mn_ref[...])
        o = jnp.maximum(pick * av + c_ref[...], 0.0)
        o_ref[...] = jnp.transpose(o, (1, 0))[None]

    return pl.pallas_call(
        body,
        grid=(n_t,),
        in_specs=[
            pl.BlockSpec((tile, c_out), lambda t: (t, 0)),
            pl.BlockSpec((tile, c_out), lambda t: (t, 0)),
            pl.BlockSpec((1, c_out), lambda t: (0, 0)),
            pl.BlockSpec((1, c_out), lambda t: (0, 0)),
        ],
        out_specs=pl.BlockSpec((1, c_out, tile),
                               lambda t: (t // per_b, 0, t % per_b)),
        out_shape=jax.ShapeDtypeStruct((B, c_out, NPOINT), jnp.float32),
    )(mx, mn, a, c)


def _bn_affine(st, g, b):
    mu = st[0] / M_ROWS
    var = jnp.maximum(st[1] / M_ROWS - mu * mu, 0.0)
    a = g / jnp.sqrt(var + EPS)
    c = b - mu * a
    return a[None, :], c[None, :]


def kernel(xyz, features, W1, g1, b1, W2, g2, b2, W3, g3, b3):
    xyz_t = jnp.transpose(xyz, (0, 2, 1))
    new_xyz = _fps(xyz_t)
    idx = _ballq(xyz_t, new_xyz)
    idx_flat = idx.reshape(M_ROWS)

    feats_t = jnp.transpose(features, (0, 2, 1))
    table = jnp.concatenate(
        [xyz, feats_t,
         jnp.zeros((B, N, D_PAD - 3 - C_IN), jnp.float32)], axis=-1)
    table = table.reshape(B * N, D_PAD)
    X = _gather_rows(table, idx_flat)

    rep_c = jnp.broadcast_to(new_xyz[:, :, None, :],
                             (B, NPOINT, NSAMPLE, 3)).reshape(M_ROWS, 3)
    C8 = jnp.concatenate([rep_c, jnp.zeros((M_ROWS, 5), jnp.float32)], axis=-1)

    c1_, c2_, c3_ = W1.shape[0], W2.shape[0], W3.shape[0]
    W1p = jnp.concatenate(
        [W1.T, jnp.zeros((D_PAD - W1.shape[1], c1_), jnp.float32)], axis=0)
    Wx8 = jnp.concatenate(
        [W1[:, :3].T, jnp.zeros((5, c1_), jnp.float32)], axis=0)

    y1, st1 = _p1(X, C8, W1p, Wx8, c1_)
    a1, k1 = _bn_affine(st1, g1, b1)
    y2, st2 = _p2(y1, a1, k1, W2.T, c1_, c2_)
    a2, k2 = _bn_affine(st2, g2, b2)
    mx, mn, st3 = _p3(y2, a2, k2, W3.T, c2_, c3_)
    a3, k3 = _bn_affine(st3, g3, b3)
    new_features = _p4(mx, mn, a3, k3, c3_)

    return (new_xyz, new_features)


# speedup vs baseline: 16.2542x; 1.3111x over previous
import functools

import jax
import jax.numpy as jnp
from jax import lax
from jax.experimental import pallas as pl
from jax.experimental.pallas import tpu as pltpu
from jax.experimental.pallas import tpu_sc as plsc

B = 4
N = 8192
C_IN = 64
NPOINT = 1024
NSAMPLE = 32
RADIUS = 0.12
EPS = 1e-5
M_ROWS = B * NPOINT * NSAMPLE
D_PAD = 128
ROW_TILE = 512
N_TILES = M_ROWS // ROW_TILE
CT_TILE = 256


def _fps_body(xyz_ref, newxyz_ref, dists_ref):
    X = xyz_ref[:, 0, :]
    Y = xyz_ref[:, 1, :]
    Z = xyz_ref[:, 2, :]
    XYZ = jnp.concatenate([X, Y, Z], axis=0)
    iota = lax.broadcasted_iota(jnp.int32, (B, N), 1)
    iota3 = lax.broadcasted_iota(jnp.int32, (3 * B, N), 1)
    dists_ref[...] = jnp.full((B, N), 1e10, jnp.float32)

    def coords_of(last):
        m3 = iota3 == jnp.concatenate([last, last, last], axis=0)
        r = jnp.sum(jnp.where(m3, XYZ, 0.0), axis=1, keepdims=True)
        return r[0:B], r[B:2 * B], r[2 * B:3 * B]

    def body(i, last):
        lx, ly, lz = coords_of(last)
        newxyz_ref[:, pl.ds(i - 1, 1), :] = jnp.concatenate(
            [lx[:, :, None], ly[:, :, None], lz[:, :, None]], axis=2)
        d = (X - lx) ** 2 + (Y - ly) ** 2 + (Z - lz) ** 2
        nd = jnp.minimum(dists_ref[...], d)
        dists_ref[...] = nd
        m = jnp.max(nd, axis=1, keepdims=True)
        nxt = jnp.min(jnp.where(nd == m, iota, N), axis=1, keepdims=True)
        return nxt.astype(jnp.int32)

    last = lax.fori_loop(1, NPOINT, body, jnp.zeros((B, 1), jnp.int32))
    lx, ly, lz = coords_of(last)
    newxyz_ref[:, NPOINT - 1:NPOINT, :] = jnp.concatenate(
        [lx[:, :, None], ly[:, :, None], lz[:, :, None]], axis=2)


def _fps(xyz_t):
    return pl.pallas_call(
        _fps_body,
        out_shape=jax.ShapeDtypeStruct((B, NPOINT, 3), jnp.float32),
        scratch_shapes=[pltpu.VMEM((B, N), jnp.float32)],
    )(xyz_t)


def _ballq_body(xyz_ref, c_ref, idx_ref, key_ref):
    t = pl.program_id(0)
    b = t // (NPOINT // CT_TILE)
    P = xyz_ref[0]
    px = xyz_ref[0, 0:1, :]
    py = xyz_ref[0, 1:2, :]
    pz = xyz_ref[0, 2:3, :]
    pn = px * px + py * py + pz * pz
    Ct = c_ref[0]
    cn = jnp.sum(Ct * Ct, axis=1, keepdims=True)
    dot = lax.dot_general(Ct, P, (((1,), (0,)), ((), ())),
                          preferred_element_type=jnp.float32)
    d2 = (cn + pn) - 2.0 * dot
    r2 = jnp.float32(RADIUS * RADIUS)
    mask = jnp.where(d2 < r2, 1, 0)

    nw = N // 16
    w = mask[:, 0:nw]
    for k in range(1, 16):
        w = w + (mask[:, k * nw:(k + 1) * nw] << k)
    key_ref[...] = w
    wiota = lax.broadcasted_iota(jnp.int32, (CT_TILE, nw), 1)
    liota = lax.broadcasted_iota(jnp.int32, (CT_TILE, NSAMPLE), 1)

    def body(j, acc):
        w = key_ref[...]
        lb = jnp.bitwise_and(w, -w)
        e = (lax.bitcast_convert_type(lb.astype(jnp.float32), jnp.int32)
             >> 23) - 127
        cand = jnp.where(w != 0, (e << 9) + wiota, N)
        m = jnp.min(cand, axis=1, keepdims=True)
        q = jnp.where(m < N, jnp.bitwise_and(m, nw - 1), -1)
        key_ref[...] = jnp.where(wiota == q, jnp.bitwise_and(w, w - 1), w)
        return jnp.where(liota == j, m, acc)

    acc = lax.fori_loop(0, NSAMPLE, body,
                        jnp.full((CT_TILE, NSAMPLE), N, jnp.int32))
    acc = jnp.where(acc == N, acc[:, 0:1], acc)
    idx_ref[...] = jnp.minimum(acc, N - 1) + b * N


def _ballq(xyz_t, new_xyz):
    n_ct = NPOINT // CT_TILE
    return pl.pallas_call(
        _ballq_body,
        grid=(B * n_ct,),
        in_specs=[
            pl.BlockSpec((1, 3, N), lambda t: (t // n_ct, 0, 0)),
            pl.BlockSpec((1, CT_TILE, 3), lambda t: (t // n_ct, t % n_ct, 0)),
        ],
        out_specs=pl.BlockSpec((CT_TILE, NSAMPLE), lambda t: (t, 0)),
        out_shape=jax.ShapeDtypeStruct((B * NPOINT, NSAMPLE), jnp.int32),
        scratch_shapes=[pltpu.VMEM((CT_TILE, N // 16), jnp.int32)],
    )(xyz_t, new_xyz)


def _gather_rows(table, idx_flat):
    info = plsc.get_sparse_core_info()
    nw = info.num_cores * info.num_subcores
    rows_per_w = M_ROWS // nw
    chunk = 512
    n_ch = rows_per_w // chunk
    mesh = plsc.VectorSubcoreMesh(core_axis_name="c", subcore_axis_name="s")

    @functools.partial(
        pl.kernel, mesh=mesh,
        out_type=jax.ShapeDtypeStruct((M_ROWS, D_PAD), jnp.float32),
        scratch_types=[
            pltpu.VMEM((chunk,), jnp.int32),
            pltpu.VMEM((chunk, D_PAD), jnp.float32),
            pltpu.SemaphoreType.DMA,
        ],
    )
    def k(table_hbm, idx_hbm, out_hbm, idx_v, rows_v, sem):
        wid = lax.axis_index("s") * info.num_cores + lax.axis_index("c")
        base = wid * rows_per_w

        def body(j, carry):
            off = base + j * chunk
            pltpu.sync_copy(idx_hbm.at[pl.ds(off, chunk)], idx_v)
            pltpu.async_copy(table_hbm.at[idx_v], rows_v, sem).wait()
            pltpu.sync_copy(rows_v, out_hbm.at[pl.ds(off, chunk)])
            return carry

        lax.fori_loop(0, n_ch, body, 0)

    return k(table, idx_flat)


def _stats_update(st_ref, y):
    t = pl.program_id(0)

    @pl.when(t == 0)
    def _():
        st_ref[...] = jnp.zeros_like(st_ref)

    s = jnp.sum(y, axis=0, keepdims=True)
    ss = jnp.sum(y * y, axis=0, keepdims=True)
    st_ref[...] += jnp.concatenate([s, ss], axis=0)


def _p1(X, C8, W1p, Wx8, c_out):
    def body(x_ref, c_ref, w_ref, wx_ref, y_ref, st_ref):
        y = jnp.dot(x_ref[...], w_ref[...], preferred_element_type=jnp.float32)
        y = y - jnp.dot(c_ref[...], wx_ref[...],
                        preferred_element_type=jnp.float32)
        y_ref[...] = y
        _stats_update(st_ref, y)

    return pl.pallas_call(
        body,
        grid=(N_TILES,),
        in_specs=[
            pl.BlockSpec((ROW_TILE, D_PAD), lambda t: (t, 0)),
            pl.BlockSpec((ROW_TILE, 8), lambda t: (t, 0)),
            pl.BlockSpec((D_PAD, c_out), lambda t: (0, 0)),
            pl.BlockSpec((8, c_out), lambda t: (0, 0)),
        ],
        out_specs=[
            pl.BlockSpec((ROW_TILE, c_out), lambda t: (t, 0)),
            pl.BlockSpec((2, c_out), lambda t: (0, 0)),
        ],
        out_shape=[
            jax.ShapeDtypeStruct((M_ROWS, c_out), jnp.float32),
            jax.ShapeDtypeStruct((2, c_out), jnp.float32),
        ],
    )(X, C8, W1p, Wx8)


def _p2(Y, a, c, Wt, c_in, c_out):
    def body(y_ref, a_ref, c_ref, w_ref, o_ref, st_ref):
        z = jnp.maximum(y_ref[...] * a_ref[...] + c_ref[...], 0.0)
        y = jnp.dot(z, w_ref[...], preferred_element_type=jnp.float32)
        o_ref[...] = y
        _stats_update(st_ref, y)

    return pl.pallas_call(
        body,
        grid=(N_TILES,),
        in_specs=[
            pl.BlockSpec((ROW_TILE, c_in), lambda t: (t, 0)),
            pl.BlockSpec((1, c_in), lambda t: (0, 0)),
            pl.BlockSpec((1, c_in), lambda t: (0, 0)),
            pl.BlockSpec((c_in, c_out), lambda t: (0, 0)),
        ],
        out_specs=[
            pl.BlockSpec((ROW_TILE, c_out), lambda t: (t, 0)),
            pl.BlockSpec((2, c_out), lambda t: (0, 0)),
        ],
        out_shape=[
            jax.ShapeDtypeStruct((M_ROWS, c_out), jnp.float32),
            jax.ShapeDtypeStruct((2, c_out), jnp.float32),
        ],
    )(Y, a, c, Wt)


def _p3(Y, a, c, Wt, c_in, c_out):
    groups = ROW_TILE // NSAMPLE

    def body(y_ref, a_ref, c_ref, w_ref, mx_ref, mn_ref, st_ref):
        z = jnp.maximum(y_ref[...] * a_ref[...] + c_ref[...], 0.0)
        y = jnp.dot(z, w_ref[...], preferred_element_type=jnp.float32)
        _stats_update(st_ref, y)
        mxs, mns = [], []
        for g in range(groups):
            blk = y[g * NSAMPLE:(g + 1) * NSAMPLE]
            mxs.append(jnp.max(blk, axis=0, keepdims=True))
            mns.append(jnp.min(blk, axis=0, keepdims=True))
        mx_ref[...] = jnp.concatenate(mxs, axis=0)
        mn_ref[...] = jnp.concatenate(mns, axis=0)

    n_grp = B * NPOINT
    return pl.pallas_call(
        body,
        grid=(N_TILES,),
        in_specs=[
            pl.BlockSpec((ROW_TILE, c_in), lambda t: (t, 0)),
            pl.BlockSpec((1, c_in), lambda t: (0, 0)),
            pl.BlockSpec((1, c_in), lambda t: (0, 0)),
            pl.BlockSpec((c_in, c_out), lambda t: (0, 0)),
        ],
        out_specs=[
            pl.BlockSpec((groups, c_out), lambda t: (t, 0)),
            pl.BlockSpec((groups, c_out), lambda t: (t, 0)),
            pl.BlockSpec((2, c_out), lambda t: (0, 0)),
        ],
        out_shape=[
            jax.ShapeDtypeStruct((n_grp, c_out), jnp.float32),
            jax.ShapeDtypeStruct((n_grp, c_out), jnp.float32),
            jax.ShapeDtypeStruct((2, c_out), jnp.float32),
        ],
    )(Y, a, c, Wt)


def _p4(mx, mn, a, c, c_out):
    tile = 512
    n_t = (B * NPOINT) // tile
    per_b = NPOINT // tile

    def body(mx_ref, mn_ref, a_ref, c_ref, o_ref):
        av = a_ref[...]
        pick = jnp.where(av >= 0.0, mx_ref[...], mn_ref[...])
        o = jnp.maximum(pick * av + c_ref[...], 0.0)
        o_ref[...] = jnp.transpose(o, (1, 0))[None]

    return pl.pallas_call(
        body,
        grid=(n_t,),
        in_specs=[
            pl.BlockSpec((tile, c_out), lambda t: (t, 0)),
            pl.BlockSpec((tile, c_out), lambda t: (t, 0)),
            pl.BlockSpec((1, c_out), lambda t: (0, 0)),
            pl.BlockSpec((1, c_out), lambda t: (0, 0)),
        ],
        out_specs=pl.BlockSpec((1, c_out, tile),
                               lambda t: (t // per_b, 0, t % per_b)),
        out_shape=jax.ShapeDtypeStruct((B, c_out, NPOINT), jnp.float32),
    )(mx, mn, a, c)


def _bn_affine(st, g, b):
    mu = st[0] / M_ROWS
    var = jnp.maximum(st[1] / M_ROWS - mu * mu, 0.0)
    a = g / jnp.sqrt(var + EPS)
    c = b - mu * a
    return a[None, :], c[None, :]


def kernel(xyz, features, W1, g1, b1, W2, g2, b2, W3, g3, b3):
    xyz_t = jnp.transpose(xyz, (0, 2, 1))
    new_xyz = _fps(xyz_t)
    idx = _ballq(xyz_t, new_xyz)
    idx_flat = idx.reshape(M_ROWS)

    feats_t = jnp.transpose(features, (0, 2, 1))
    table = jnp.concatenate(
        [xyz, feats_t,
         jnp.zeros((B, N, D_PAD - 3 - C_IN), jnp.float32)], axis=-1)
    table = table.reshape(B * N, D_PAD)
    X = _gather_rows(table, idx_flat)

    rep_c = jnp.broadcast_to(new_xyz[:, :, None, :],
                             (B, NPOINT, NSAMPLE, 3)).reshape(M_ROWS, 3)
    C8 = jnp.concatenate([rep_c, jnp.zeros((M_ROWS, 5), jnp.float32)], axis=-1)

    c1_, c2_, c3_ = W1.shape[0], W2.shape[0], W3.shape[0]
    W1p = jnp.concatenate(
        [W1.T, jnp.zeros((D_PAD - W1.shape[1], c1_), jnp.float32)], axis=0)
    Wx8 = jnp.concatenate(
        [W1[:, :3].T, jnp.zeros((5, c1_), jnp.float32)], axis=0)

    y1, st1 = _p1(X, C8, W1p, Wx8, c1_)
    a1, k1 = _bn_affine(st1, g1, b1)
    y2, st2 = _p2(y1, a1, k1, W2.T, c1_, c2_)
    a2, k2 = _bn_affine(st2, g2, b2)
    mx, mn, st3 = _p3(y2, a2, k2, W3.T, c2_, c3_)
    a3, k3 = _bn_affine(st3, g3, b3)
    new_features = _p4(mx, mn, a3, k3, c3_)

    return (new_xyz, new_features)
